# Initial kernel scaffold; baseline (speedup 1.0000x reference)
import jax
import jax.numpy as jnp
from jax.experimental import pallas as pl


def kernel(center, positive_context, negative_context, batch_size, center_table, context_table):
    raise NotImplementedError("write your pallas kernel here")


# speedup vs baseline: 2.9277x; 2.9277x over previous
import functools

import jax
import jax.numpy as jnp
from jax import lax
from jax.experimental import pallas as pl
from jax.experimental.pallas import tpu as pltpu
from jax.experimental.pallas import tpu_sc as plsc

DIM = 64
K = 10
LANES = 16
NQ = DIM // LANES
CHUNK = 128


def _sc_partials(center2d, pos2d, neg2d, center_table, context_table,
                 n_workers, n_chunks, b):
    bpw = b // n_workers
    crows_pw = bpw // CHUNK
    nrows_pw = bpw * K // CHUNK
    nrows_pc = CHUNK * K // CHUNK
    mesh = plsc.VectorSubcoreMesh(core_axis_name="c", subcore_axis_name="s")
    nc = 2

    @functools.partial(
        pl.kernel,
        out_type=[
            jax.ShapeDtypeStruct((b, LANES), jnp.float32),
            jax.ShapeDtypeStruct((b, LANES), jnp.float32),
        ],
        mesh=mesh,
        compiler_params=pltpu.CompilerParams(use_tc_tiling_on_sc=False),
        scratch_types=[
            pltpu.VMEM((crows_pw, CHUNK), jnp.int32),
            pltpu.VMEM((crows_pw, CHUNK), jnp.int32),
            pltpu.VMEM((nrows_pw, CHUNK), jnp.int32),
            pltpu.VMEM((CHUNK, DIM), jnp.float32),
            pltpu.VMEM((CHUNK, DIM), jnp.float32),
            pltpu.VMEM((CHUNK * K, DIM), jnp.float32),
            pltpu.VMEM((CHUNK, LANES), jnp.float32),
            pltpu.VMEM((CHUNK, LANES), jnp.float32),
            pltpu.SemaphoreType.DMA,
        ],
    )
    def sc_kern(cidx_hbm, pidx_hbm, nidx_hbm, ctab_hbm, xtab_hbm,
                pos_out, neg_out,
                cidx_v, pidx_v, nidx_v, crow, prow, nrow, posb, negb, sem):
        wid = lax.axis_index("s") * nc + lax.axis_index("c")
        pltpu.sync_copy(cidx_hbm.at[pl.ds(wid * crows_pw, crows_pw)], cidx_v)
        pltpu.sync_copy(pidx_hbm.at[pl.ds(wid * crows_pw, crows_pw)], pidx_v)
        pltpu.sync_copy(nidx_hbm.at[pl.ds(wid * nrows_pw, nrows_pw)], nidx_v)

        for m in range(n_chunks):
            copies = [
                pltpu.async_copy(ctab_hbm.at[cidx_v.at[m]], crow, sem),
                pltpu.async_copy(xtab_hbm.at[pidx_v.at[m]], prow, sem),
            ]
            for j in range(nrows_pc):
                copies.append(pltpu.async_copy(
                    xtab_hbm.at[nidx_v.at[m * nrows_pc + j]],
                    nrow.at[pl.ds(j * CHUNK, CHUNK)], sem))
            for c in copies:
                c.wait()

            def body(e, _):
                cs = [crow[e, pl.ds(q * LANES, LANES)] for q in range(NQ)]
                ps = [prow[e, pl.ds(q * LANES, LANES)] for q in range(NQ)]
                pacc = cs[0] * ps[0]
                for q in range(1, NQ):
                    pacc = pacc + cs[q] * ps[q]
                nb = e * K
                nacc = None
                for q in range(NQ):
                    ns = nrow[nb, pl.ds(q * LANES, LANES)]
                    for j in range(1, K):
                        ns = ns + nrow[nb + j, pl.ds(q * LANES, LANES)]
                    t = ns * cs[q]
                    nacc = t if nacc is None else nacc + t
                posb[e, :] = pacc
                negb[e, :] = nacc
                return _

            lax.fori_loop(0, CHUNK, body, 0, unroll=2)
            base = wid * bpw + m * CHUNK
            pltpu.sync_copy(posb, pos_out.at[pl.ds(base, CHUNK)])
            pltpu.sync_copy(negb, neg_out.at[pl.ds(base, CHUNK)])

    return sc_kern(center2d, pos2d, neg2d, center_table, context_table)


def _log_sigmoid(x):
    return jnp.minimum(x, 0.0) - jnp.log1p(jnp.exp(-jnp.abs(x)))


def _tc_reduce_body(pos_ref, neg_ref, out_ref):
    pd = jnp.sum(pos_ref[...], axis=1, keepdims=True)
    nd = jnp.sum(neg_ref[...], axis=1, keepdims=True)
    tot = jnp.sum(_log_sigmoid(pd)) + jnp.sum(_log_sigmoid(-nd))
    out_ref[0, 0] = tot


def kernel(center, positive_context, negative_context, batch_size,
           center_table, context_table):
    b = center.shape[0]
    n_workers = 32
    n_chunks = b // n_workers // CHUNK
    center2d = center.astype(jnp.int32).reshape(b // CHUNK, CHUNK)
    pos2d = positive_context.astype(jnp.int32).reshape(b // CHUNK, CHUNK)
    neg2d = negative_context.astype(jnp.int32).reshape(b * K // CHUNK, CHUNK)

    pos_part, neg_part = _sc_partials(
        center2d, pos2d, neg2d, center_table, context_table,
        n_workers, n_chunks, b)

    tot = pl.pallas_call(
        _tc_reduce_body,
        out_shape=jax.ShapeDtypeStruct((1, 1), jnp.float32),
        out_specs=pl.BlockSpec(memory_space=pltpu.SMEM),
    )(pos_part, neg_part)
    return -tot[0, 0] / batch_size
